# Initial kernel scaffold; baseline (speedup 1.0000x reference)
import jax
import jax.numpy as jnp
from jax.experimental import pallas as pl


def kernel(input_ids, table, W, b):
    raise NotImplementedError("write your pallas kernel here")


# speedup vs baseline: 67.9429x; 67.9429x over previous
import functools

import jax
import jax.numpy as jnp
from jax import lax
from jax.experimental import pallas as pl
from jax.experimental.pallas import tpu as pltpu
from jax.experimental.pallas import tpu_sc as plsc

VOCAB_N = 1000
EMBED_N = 128
OUT_N = 2
BATCH_N = 4096
SEQ_N = 200
VPAD = 1024
LANES = 16
UNROLL = 8
L_STEPS = SEQ_N // UNROLL


def _project_body(w_ref, t_ref, b_ref, o_ref):
    p = lax.dot_general(
        w_ref[...], t_ref[...],
        dimension_numbers=(((1,), (1,)), ((), ())),
        preferred_element_type=jnp.float32,
    )
    o_ref[...] = (p + b_ref[...]) * (1.0 / SEQ_N)


def _project(table, w, b):
    return pl.pallas_call(
        _project_body,
        out_shape=jax.ShapeDtypeStruct((OUT_N, VOCAB_N), jnp.float32),
    )(w, table, b.reshape(OUT_N, 1))


@functools.lru_cache(maxsize=None)
def _make_sc_pool(nc, ns):
    nw = nc * ns
    rows_per_w = BATCH_N // nw
    idx_per_w = rows_per_w * SEQ_N
    groups = rows_per_w // LANES

    mesh = plsc.VectorSubcoreMesh(core_axis_name="c", subcore_axis_name="s")

    @functools.partial(
        pl.kernel,
        mesh=mesh,
        compiler_params=pltpu.CompilerParams(needs_layout_passes=False),
        out_type=(
            jax.ShapeDtypeStruct((BATCH_N,), jnp.float32),
            jax.ShapeDtypeStruct((BATCH_N,), jnp.float32),
        ),
        scratch_types=[
            pltpu.VMEM((VPAD,), jnp.float32),
            pltpu.VMEM((VPAD,), jnp.float32),
            pltpu.VMEM((idx_per_w,), jnp.int32),
            pltpu.VMEM((rows_per_w,), jnp.float32),
            pltpu.VMEM((rows_per_w,), jnp.float32),
        ],
    )
    def sc_pool(p_hbm, ids_hbm, o0_hbm, o1_hbm, p0_v, p1_v, idx_v, o0_v, o1_v):
        wid = lax.axis_index("s") * nc + lax.axis_index("c")
        base = wid * idx_per_w
        pltpu.sync_copy(p_hbm.at[0], p0_v)
        pltpu.sync_copy(p_hbm.at[1], p1_v)
        pltpu.sync_copy(ids_hbm.at[pl.ds(base, idx_per_w)], idx_v)

        lane_off = lax.iota(jnp.int32, LANES) * SEQ_N

        def group_body(g, carry):
            gbase = g * (LANES * SEQ_N)

            def l_body(s, accs):
                a0, a1 = accs
                for u in range(UNROLL):
                    addrs = lane_off + (gbase + s * UNROLL + u)
                    idx = plsc.load_gather(idx_v, [addrs])
                    a0 = a0 + plsc.load_gather(p0_v, [idx])
                    a1 = a1 + plsc.load_gather(p1_v, [idx])
                return a0, a1

            zero = jnp.zeros((LANES,), jnp.float32)
            a0, a1 = lax.fori_loop(0, L_STEPS, l_body, (zero, zero))
            o0_v[pl.ds(g * LANES, LANES)] = a0
            o1_v[pl.ds(g * LANES, LANES)] = a1
            return carry

        lax.fori_loop(0, groups, group_body, 0)
        pltpu.sync_copy(o0_v, o0_hbm.at[pl.ds(wid * rows_per_w, rows_per_w)])
        pltpu.sync_copy(o1_v, o1_hbm.at[pl.ds(wid * rows_per_w, rows_per_w)])

    return sc_pool


def kernel(input_ids, table, W, b):
    p = _project(table, W, b)
    p = jnp.pad(p, ((0, 0), (0, VPAD - VOCAB_N)))
    ids_flat = input_ids.reshape(-1).astype(jnp.int32)
    info = plsc.get_sparse_core_info()
    sc_pool = _make_sc_pool(info.num_cores, info.num_subcores)
    out0, out1 = sc_pool(p, ids_flat)
    return jnp.stack([out0, out1], axis=-1)
